# Initial kernel scaffold; baseline (speedup 1.0000x reference)
import functools

import jax
import jax.numpy as jnp
from jax import lax
from jax.experimental import pallas as pl
from jax.experimental.pallas import tpu as pltpu
from jax.experimental.pallas import tpu_sc as plsc

DIM = 32
B = 16384 * 50
NC, NS = 2, 16
NW = NC * NS
BPW = B // NW
SUB = 128
K = 8
CHUNK = SUB * K
NCHUNK = BPW // CHUNK
ROWS_PER_SUBBLK = B // SUB


def _make_kernel():
    mesh = plsc.VectorSubcoreMesh(core_axis_name="c", subcore_axis_name="s")

    @functools.partial(
        pl.kernel,
        mesh=mesh,
        out_type=jax.ShapeDtypeStruct((B, DIM), jnp.float32),
        scratch_types=[
            pltpu.VMEM((K, SUB), jnp.int32),
            pltpu.VMEM((CHUNK, DIM), jnp.float32),
            pltpu.SemaphoreType.DMA,
        ],
    )
    def gather_kernel(idx_hbm, table_hbm, out_hbm, idx_v, rows_v, sem):
        wid = lax.axis_index("s") * NC + lax.axis_index("c")
        row0 = wid * (BPW // SUB)

        def body(g, carry):
            r = row0 + g * K
            pltpu.sync_copy(idx_hbm.at[pl.ds(r, K)], idx_v)
            waits = []
            for j in range(K):
                waits.append(
                    pltpu.async_copy(
                        table_hbm.at[idx_v.at[j]],
                        rows_v.at[pl.ds(j * SUB, SUB)],
                        sem,
                    )
                )
            for w in waits:
                w.wait()
            pltpu.sync_copy(rows_v, out_hbm.at[pl.ds(r * SUB, CHUNK)])
            return carry

        lax.fori_loop(0, NCHUNK, body, 0)

    return gather_kernel


_gather = _make_kernel()


def kernel(inputs, table):
    idx2d = inputs.reshape(ROWS_PER_SUBBLK, SUB).astype(jnp.int32)
    out = _gather(idx2d, table)
    return out.reshape(inputs.shape + (DIM,))


# speedup vs baseline: 1.0943x; 1.0943x over previous
import functools

import jax
import jax.numpy as jnp
from jax import lax
from jax.experimental import pallas as pl
from jax.experimental.pallas import tpu as pltpu
from jax.experimental.pallas import tpu_sc as plsc

DIM = 32
B = 16384 * 50
NC, NS = 2, 16
NW = NC * NS
BPW = B // NW
SUB = 128
K = 8
CHUNK = SUB * K
NCHUNK = BPW // CHUNK
ROWS_PER_SUBBLK = B // SUB


def _make_kernel():
    mesh = plsc.VectorSubcoreMesh(core_axis_name="c", subcore_axis_name="s")

    @functools.partial(
        pl.kernel,
        mesh=mesh,
        out_type=jax.ShapeDtypeStruct((B, DIM), jnp.float32),
        scratch_types=[
            pltpu.VMEM((K, SUB), jnp.int32),
            pltpu.VMEM((CHUNK, DIM), jnp.float32),
            pltpu.SemaphoreType.DMA,
        ],
        compiler_params=pltpu.CompilerParams(use_tc_tiling_on_sc=False),
    )
    def gather_kernel(idx_hbm, table_hbm, out_hbm, idx_v, rows_v, sem):
        wid = lax.axis_index("s") * NC + lax.axis_index("c")
        row0 = wid * (BPW // SUB)

        def body(g, carry):
            r = row0 + g * K
            pltpu.sync_copy(idx_hbm.at[pl.ds(r, K)], idx_v)
            waits = []
            for j in range(K):
                waits.append(
                    pltpu.async_copy(
                        table_hbm.at[idx_v.at[j]],
                        rows_v.at[pl.ds(j * SUB, SUB)],
                        sem,
                    )
                )
            for w in waits:
                w.wait()
            pltpu.sync_copy(rows_v, out_hbm.at[pl.ds(r * SUB, CHUNK)])
            return carry

        lax.fori_loop(0, NCHUNK, body, 0)

    return gather_kernel


_gather = _make_kernel()


def kernel(inputs, table):
    idx2d = inputs.reshape(ROWS_PER_SUBBLK, SUB).astype(jnp.int32)
    out = _gather(idx2d, table)
    return out.reshape(inputs.shape + (DIM,))


# speedup vs baseline: 1.1135x; 1.0175x over previous
import functools

import jax
import jax.numpy as jnp
from jax import lax
from jax.experimental import pallas as pl
from jax.experimental.pallas import tpu as pltpu
from jax.experimental.pallas import tpu_sc as plsc

DIM = 32
B = 16384 * 50
NC, NS = 2, 16
NW = NC * NS
BPW = B // NW
SUB = 128
K = 4
CHUNK = SUB * K
NCH = BPW // CHUNK
IROWS = BPW // SUB
NBUF = 3


def _make_kernel():
    mesh = plsc.VectorSubcoreMesh(core_axis_name="c", subcore_axis_name="s")

    @functools.partial(
        pl.kernel,
        mesh=mesh,
        out_type=jax.ShapeDtypeStruct((B, DIM), jnp.float32),
        scratch_types=[
            pltpu.VMEM((IROWS, SUB), jnp.int32),
            pltpu.VMEM((NBUF, CHUNK, DIM), jnp.float32),
            pltpu.SemaphoreType.DMA,
            pltpu.SemaphoreType.DMA,
            pltpu.SemaphoreType.DMA,
            pltpu.SemaphoreType.DMA,
            pltpu.SemaphoreType.DMA,
            pltpu.SemaphoreType.DMA,
        ],
        compiler_params=pltpu.CompilerParams(use_tc_tiling_on_sc=False),
    )
    def gather_kernel(idx_hbm, table_hbm, out_hbm, idx_v, rows_v,
                      sg0, sg1, sg2, ss0, ss1, ss2):
        sg = (sg0, sg1, sg2)
        ss = (ss0, ss1, ss2)
        wid = lax.axis_index("s") * NC + lax.axis_index("c")
        irow0 = wid * IROWS
        orow0 = wid * BPW

        pltpu.sync_copy(idx_hbm.at[pl.ds(irow0, IROWS)], idx_v)

        def fire_gather(c, b):
            for j in range(K):
                pltpu.async_copy(
                    table_hbm.at[idx_v.at[c * K + j]],
                    rows_v.at[b, pl.ds(j * SUB, SUB)],
                    sg[b],
                )

        def wait_gather(b):
            pltpu.make_async_copy(
                table_hbm.at[pl.ds(0, CHUNK)], rows_v.at[b], sg[b]
            ).wait()

        def fire_scatter(c, b):
            pltpu.async_copy(
                rows_v.at[b], out_hbm.at[pl.ds(orow0 + c * CHUNK, CHUNK)], ss[b]
            )

        def wait_scatter(b):
            pltpu.make_async_copy(
                rows_v.at[b], out_hbm.at[pl.ds(0, CHUNK)], ss[b]
            ).wait()

        def step(c, b, first=False):
            wait_gather(b)
            fire_scatter(c, b)
            b2 = (b + 2) % NBUF
            if not first:
                wait_scatter(b2)
            fire_gather(c + 2, b2)

        fire_gather(0, 0)
        fire_gather(1, 1)
        step(0, 0, first=True)
        step(1, 1)
        step(2, 2)

        def body(g, carry):
            c0 = g * NBUF
            step(c0, 0)
            step(c0 + 1, 1)
            step(c0 + 2, 2)
            return carry

        lax.fori_loop(1, (NCH - 2) // NBUF, body, 0)

        wait_gather(0)
        fire_scatter(NCH - 2, 0)
        wait_gather(1)
        fire_scatter(NCH - 1, 1)
        wait_scatter(2)
        wait_scatter(0)
        wait_scatter(1)

    return gather_kernel


_gather = _make_kernel()


def kernel(inputs, table):
    idx2d = inputs.reshape(B // SUB, SUB).astype(jnp.int32)
    out = _gather(idx2d, table)
    return out.reshape(inputs.shape + (DIM,))


# speedup vs baseline: 1.6174x; 1.4525x over previous
import functools

import jax
import jax.numpy as jnp
from jax import lax
from jax.experimental import pallas as pl
from jax.experimental.pallas import tpu as pltpu
from jax.experimental.pallas import tpu_sc as plsc

DIM = 32
NB = 16384
NS = 50
B = NB * NS
NCORE, NSUB = 2, 16
NW = NCORE * NSUB
BPW = NB // NW
KCH = 4
BC = BPW // KCH
SCH = 5
NU = NS // SCH
TRO = DIM // 8


def _make_kernel():
    mesh = plsc.VectorSubcoreMesh(core_axis_name="c", subcore_axis_name="s")

    @functools.partial(
        pl.kernel,
        mesh=mesh,
        out_type=jax.ShapeDtypeStruct((NS, TRO, NB // BC, 8, BC), jnp.float32),
        scratch_types=[
            pltpu.VMEM((BC * NS,), jnp.int32),
            pltpu.VMEM((NS, BC), jnp.int32),
            pltpu.VMEM((2, SCH * BC, DIM), jnp.float32),
            pltpu.VMEM((2, SCH, TRO, 1, 8, BC), jnp.float32),
            pltpu.SemaphoreType.DMA,
            pltpu.SemaphoreType.DMA,
            pltpu.SemaphoreType.DMA,
            pltpu.SemaphoreType.DMA,
        ],
        compiler_params=pltpu.CompilerParams(
            use_tc_tiling_on_sc=False, needs_layout_passes=False
        ),
    )
    def gather_kernel(idx_hbm, table_hbm, out_hbm, ibuf, ibufT, gbuf, obuf,
                      g0, g1, o0, o1):
        gsem = (g0, g1)
        osem = (o0, o1)
        wid = lax.axis_index("s") * NCORE + lax.axis_index("c")
        iota16 = lax.iota(jnp.int32, 16)
        iota16x50 = iota16 * NS

        def kbody(k, carry):
            base = (wid * BPW + k * BC) * NS
            tc = wid * KCH + k
            pltpu.sync_copy(idx_hbm.at[pl.ds(base, BC * NS)], ibuf)

            def idxt_body(s, c):
                for g in range(8):
                    rows = iota16x50 + (g * 16 * NS + s)
                    ibufT[s, pl.ds(g * 16, 16)] = plsc.load_gather(ibuf, [rows])
                return c

            lax.fori_loop(0, NS, idxt_body, 0)

            def fire_g(u, b):
                for sl in range(SCH):
                    pltpu.async_copy(
                        table_hbm.at[ibufT.at[u * SCH + sl]],
                        gbuf.at[b, pl.ds(sl * BC, BC)],
                        gsem[b],
                    )

            def wait_g(b):
                pltpu.make_async_copy(
                    table_hbm.at[pl.ds(0, SCH * BC)], gbuf.at[b], gsem[b]
                ).wait()

            def fire_o(u, b):
                pltpu.async_copy(
                    obuf.at[b],
                    out_hbm.at[pl.ds(u * SCH, SCH), :, pl.ds(tc, 1)],
                    osem[b],
                )

            def wait_o(u, b):
                pltpu.make_async_copy(
                    obuf.at[b],
                    out_hbm.at[pl.ds(u * SCH, SCH), :, pl.ds(tc, 1)],
                    osem[b],
                ).wait()

            def transpose(b):
                def tbody(t, c):
                    s_l = t // DIM
                    d = t - s_l * DIM
                    tr = d // 8
                    di = d - tr * 8
                    cols = jnp.zeros((16,), jnp.int32) + d
                    rbase = s_l * BC
                    for g in range(8):
                        rows = iota16 + (rbase + g * 16)
                        v = plsc.load_gather(gbuf.at[b], [rows, cols])
                        obuf[b, s_l, tr, 0, di, pl.ds(g * 16, 16)] = v
                    return c

                lax.fori_loop(0, SCH * DIM, tbody, 0)

            fire_g(0, 0)
            for u in range(NU):
                if u + 1 < NU:
                    fire_g(u + 1, (u + 1) % 2)
                wait_g(u % 2)
                if u >= 2:
                    wait_o(u - 2, u % 2)
                transpose(u % 2)
                fire_o(u, u % 2)
            wait_o(NU - 2, 0)
            wait_o(NU - 1, 1)
            return carry

        lax.fori_loop(0, KCH, kbody, 0)

    return gather_kernel


_gather = _make_kernel()


def kernel(inputs, table):
    idx_flat = inputs.reshape(B).astype(jnp.int32)
    out5 = _gather(idx_flat, table)
    t = out5.transpose(2, 4, 0, 1, 3)
    return t.reshape(NB, NS, DIM)


# speedup vs baseline: 2.0593x; 1.2733x over previous
import functools

import jax
import jax.numpy as jnp
from jax import lax
from jax.experimental import pallas as pl
from jax.experimental.pallas import tpu as pltpu
from jax.experimental.pallas import tpu_sc as plsc

DIM = 32
NB = 16384
NS = 50
B = NB * NS
NCORE, NSUB = 2, 16
NW = NCORE * NSUB
BPW = NB // NW
KCH = 4
BC = BPW // KCH
SCH = 5
NU = NS // SCH
TRO = DIM // 8


def _make_kernel():
    mesh = plsc.VectorSubcoreMesh(core_axis_name="c", subcore_axis_name="s")

    @functools.partial(
        pl.kernel,
        mesh=mesh,
        out_type=jax.ShapeDtypeStruct((NS, TRO, NB // BC, 8, BC), jnp.float32),
        scratch_types=[
            pltpu.VMEM((BC * NS,), jnp.int32),
            pltpu.VMEM((NS, BC), jnp.int32),
            pltpu.VMEM((2, SCH * BC, DIM), jnp.float32),
            pltpu.VMEM((2, SCH, TRO, 1, 8, BC), jnp.float32),
            pltpu.SemaphoreType.DMA,
            pltpu.SemaphoreType.DMA,
            pltpu.SemaphoreType.DMA,
            pltpu.SemaphoreType.DMA,
        ],
        compiler_params=pltpu.CompilerParams(
            use_tc_tiling_on_sc=False, needs_layout_passes=False
        ),
    )
    def gather_kernel(idx_hbm, table_hbm, out_hbm, ibuf, ibufT, gbuf, obuf,
                      g0, g1, o0, o1):
        gsem = (g0, g1)
        osem = (o0, o1)
        wid = lax.axis_index("s") * NCORE + lax.axis_index("c")
        iota16 = lax.iota(jnp.int32, 16)
        iota16x50 = iota16 * NS
        zeros16 = jnp.zeros((16,), jnp.int32)
        tr0_c = iota16 // 8
        tr1_c = tr0_c + 2
        di_c = lax.rem(iota16, jnp.full((16,), 8, jnp.int32))

        def kbody(k, carry):
            base = (wid * BPW + k * BC) * NS
            tc = wid * KCH + k
            pltpu.sync_copy(idx_hbm.at[pl.ds(base, BC * NS)], ibuf)

            @plsc.parallel_loop(0, NS, unroll=2)
            def _idxt(s):
                for g in range(8):
                    rows = iota16x50 + (g * 16 * NS + s)
                    ibufT[s, pl.ds(g * 16, 16)] = plsc.load_gather(ibuf, [rows])

            def fire_g(u, b):
                for sl in range(SCH):
                    pltpu.async_copy(
                        table_hbm.at[ibufT.at[u * SCH + sl]],
                        gbuf.at[b, pl.ds(sl * BC, BC)],
                        gsem[b],
                    )

            def wait_g(b):
                pltpu.make_async_copy(
                    table_hbm.at[pl.ds(0, SCH * BC)], gbuf.at[b], gsem[b]
                ).wait()

            def fire_o(u, b):
                pltpu.async_copy(
                    obuf.at[b],
                    out_hbm.at[pl.ds(u * SCH, SCH), :, pl.ds(tc, 1)],
                    osem[b],
                )

            def wait_o(u, b):
                pltpu.make_async_copy(
                    obuf.at[b],
                    out_hbm.at[pl.ds(u * SCH, SCH), :, pl.ds(tc, 1)],
                    osem[b],
                ).wait()

            def transpose(b):
                for s_l in range(SCH):
                    s_c = zeros16 + s_l
                    rbase = s_l * BC

                    @plsc.parallel_loop(0, BC, unroll=8)
                    def _row(bi):
                        bi_c = zeros16 + bi
                        v0 = gbuf[b, rbase + bi, pl.ds(0, 16)]
                        v1 = gbuf[b, rbase + bi, pl.ds(16, 16)]
                        plsc.store_scatter(
                            obuf.at[b], [s_c, tr0_c, zeros16, di_c, bi_c], v0
                        )
                        plsc.store_scatter(
                            obuf.at[b], [s_c, tr1_c, zeros16, di_c, bi_c], v1
                        )

            fire_g(0, 0)
            for u in range(NU):
                if u + 1 < NU:
                    fire_g(u + 1, (u + 1) % 2)
                wait_g(u % 2)
                if u >= 2:
                    wait_o(u - 2, u % 2)
                transpose(u % 2)
                fire_o(u, u % 2)
            wait_o(NU - 2, 0)
            wait_o(NU - 1, 1)
            return carry

        lax.fori_loop(0, KCH, kbody, 0)

    return gather_kernel


_gather = _make_kernel()


def kernel(inputs, table):
    idx_flat = inputs.reshape(B).astype(jnp.int32)
    out5 = _gather(idx_flat, table)
    t = out5.transpose(2, 4, 0, 1, 3)
    return t.reshape(NB, NS, DIM)
